# baseline (speedup 1.0000x reference)
import dataclasses
import functools

import jax
import jax.numpy as jnp
from jax import lax
from jax.experimental import pallas as pl
from jax.experimental.pallas import tpu as pltpu
from jax.experimental.pallas import tpu_sc as plsc

N = 10000
E = 320000
D = 128

NC = 2
NS = 16
NW = NC * NS

_F32 = jnp.float32
_PREC = lax.Precision.DEFAULT


def _node_in_body(x_ref, w1a_ref, w1b_ref, b1_ref, p_ref, q_ref):
    xb = x_ref[...]
    p_ref[...] = jnp.dot(xb, w1a_ref[...], preferred_element_type=_F32, precision=_PREC) + b1_ref[...]
    q_ref[...] = jnp.dot(xb, w1b_ref[...], preferred_element_type=_F32, precision=_PREC)


_node_in = pl.pallas_call(
    _node_in_body,
    grid=(5,),
    in_specs=[
        pl.BlockSpec((2000, D), lambda i: (i, 0)),
        pl.BlockSpec((D, D), lambda i: (0, 0)),
        pl.BlockSpec((D, D), lambda i: (0, 0)),
        pl.BlockSpec((1, D), lambda i: (0, 0)),
    ],
    out_specs=[
        pl.BlockSpec((2000, D), lambda i: (i, 0)),
        pl.BlockSpec((2000, D), lambda i: (i, 0)),
    ],
    out_shape=[
        jax.ShapeDtypeStruct((N, D), _F32),
        jax.ShapeDtypeStruct((N, D), _F32),
    ],
)


def _edge_mlp_body(h_ref, w2_ref, b2_ref, m_ref):
    h = h_ref[...]
    m_ref[...] = jnp.maximum(
        jnp.dot(h, w2_ref[...], preferred_element_type=_F32, precision=_PREC) + b2_ref[...], 0.0)


def _make_edge_mlp(e_rows):
    return pl.pallas_call(
        _edge_mlp_body,
        grid=(e_rows // 6400,),
        in_specs=[
            pl.BlockSpec((6400, D), lambda i: (i, 0)),
            pl.BlockSpec((D, D), lambda i: (0, 0)),
            pl.BlockSpec((1, D), lambda i: (0, 0)),
        ],
        out_specs=pl.BlockSpec((6400, D), lambda i: (i, 0)),
        out_shape=jax.ShapeDtypeStruct((e_rows, D), _F32),
    )


def _update_body(xx_ref, a0_ref, a1_ref,
                 wu1_ref, bu1_ref, wu2_ref, bu2_ref, wu3_ref, bu3_ref,
                 w1a_ref, w1b_ref, b1_ref, out_ref, p_ref, q_ref):
    agg = a0_ref[...] + a1_ref[...]
    u = jnp.maximum(
        jnp.dot(agg, wu1_ref[...], preferred_element_type=_F32, precision=_PREC) + bu1_ref[...], 0.0)
    u = jnp.maximum(
        jnp.dot(u, wu2_ref[...], preferred_element_type=_F32, precision=_PREC) + bu2_ref[...], 0.0)
    u = jnp.dot(u, wu3_ref[...], preferred_element_type=_F32, precision=_PREC) + bu3_ref[...]
    o = xx_ref[...] + u
    out_ref[...] = o
    p_ref[...] = jnp.dot(o, w1a_ref[...], preferred_element_type=_F32, precision=_PREC) + b1_ref[...]
    q_ref[...] = jnp.dot(o, w1b_ref[...], preferred_element_type=_F32, precision=_PREC)


_update = pl.pallas_call(
    _update_body,
    grid=(5,),
    in_specs=[pl.BlockSpec((2000, D), lambda i: (i, 0))] * 3
    + [pl.BlockSpec((D, D), lambda i: (0, 0)), pl.BlockSpec((1, D), lambda i: (0, 0))] * 3
    + [pl.BlockSpec((D, D), lambda i: (0, 0)),
       pl.BlockSpec((D, D), lambda i: (0, 0)),
       pl.BlockSpec((1, D), lambda i: (0, 0))],
    out_specs=[
        pl.BlockSpec((2000, D), lambda i: (i, 0)),
        pl.BlockSpec((2000, D), lambda i: (i, 0)),
        pl.BlockSpec((2000, D), lambda i: (i, 0)),
    ],
    out_shape=[
        jax.ShapeDtypeStruct((N, D), _F32),
        jax.ShapeDtypeStruct((N, D), _F32),
        jax.ShapeDtypeStruct((N, D), _F32),
    ],
)


_MESH = plsc.VectorSubcoreMesh(core_axis_name="c", subcore_axis_name="s")

_SC_CP = pltpu.CompilerParams()
if "needs_layout_passes" in pltpu.CompilerParams.__dataclass_fields__:
    _SC_CP = dataclasses.replace(_SC_CP, needs_layout_passes=False)


def _make_gather(e_edges, k):
    epw = e_edges // NW
    nchunk = epw // k
    assert nchunk == 125 and k % 8 == 0

    @functools.partial(
        pl.kernel,
        out_type=jax.ShapeDtypeStruct((e_edges, D), _F32),
        mesh=_MESH,
        compiler_params=_SC_CP,
        scratch_types=[
            pltpu.VMEM((nchunk, k), jnp.int32),
            pltpu.VMEM((nchunk, k), jnp.int32),
            pltpu.VMEM((3, k, D), _F32),
            pltpu.VMEM((3, k, D), _F32),
            pltpu.VMEM((2, k, D), _F32),
            pltpu.SemaphoreType.DMA,
            pltpu.SemaphoreType.DMA,
            pltpu.SemaphoreType.DMA,
            pltpu.SemaphoreType.DMA,
            pltpu.SemaphoreType.DMA,
        ],
    )
    def gather(p_hbm, q_hbm, dst_hbm, src_hbm, h_hbm,
               di, si, pb, qb, hb, gs0, gs1, gs2, ws0, ws1):
        wid = lax.axis_index("s") * NC + lax.axis_index("c")
        base = wid * epw
        gs = (gs0, gs1, gs2)
        ws = (ws0, ws1)

        pltpu.sync_copy(dst_hbm.at[wid], di)
        pltpu.sync_copy(src_hbm.at[wid], si)

        def issue(ci, b3):
            pltpu.async_copy(p_hbm.at[di.at[ci]], pb.at[b3], gs[b3])
            pltpu.async_copy(q_hbm.at[si.at[ci]], qb.at[b3], gs[b3])

        def gwait(b3):
            pltpu.make_async_copy(p_hbm.at[di.at[0]], pb.at[b3], gs[b3]).wait()
            pltpu.make_async_copy(q_hbm.at[si.at[0]], qb.at[b3], gs[b3]).wait()

        def wbwait(b2):
            pltpu.make_async_copy(hb.at[b2], h_hbm.at[pl.ds(0, k)], ws[b2]).wait()

        def do_chunk(ci, b3, b2, guard_wb, do_prefetch):
            gwait(b3)
            if guard_wb:
                @pl.when(ci >= 2)
                def _():
                    wbwait(b2)
            else:
                wbwait(b2)

            def row(i, c2):
                for c in range(D // 16):
                    sl = pl.ds(c * 16, 16)
                    hb[b2, i, sl] = jnp.maximum(pb[b3, i, sl] + qb[b3, i, sl], 0.0)
                return c2

            lax.fori_loop(0, k, row, 0)
            pltpu.async_copy(hb.at[b2], h_hbm.at[pl.ds(base + ci * k, k)], ws[b2])
            if do_prefetch:
                issue(ci + 3, b3)

        for c0 in range(3):
            issue(c0, c0)

        def body(j, carry):
            ci0 = j * 6
            for u in range(6):
                do_chunk(ci0 + u, u % 3, u % 2, guard_wb=True, do_prefetch=True)
            return carry

        lax.fori_loop(0, 120 // 6, body, 0)

        for ci in range(120, nchunk):
            do_chunk(ci, ci % 3, ci % 2, guard_wb=False,
                     do_prefetch=(ci + 3 < nchunk))

        wbwait(1)
        wbwait(0)

    return gather


_ZK = 40
_NZ = N // _ZK
_CH = 200
_NCH = N // _CH


def _make_scatter(e_edges, k):
    epw = e_edges // NW
    nchunk = epw // k
    assert nchunk == 125 and k % 8 == 0

    @functools.partial(
        pl.kernel,
        out_type=jax.ShapeDtypeStruct((NC, N, D), _F32),
        mesh=_MESH,
        compiler_params=_SC_CP,
        scratch_types=[
            pltpu.VMEM((4, k), jnp.int32),
            pltpu.VMEM((4, k, D), _F32),
            pltpu.VMEM((_ZK, D), _F32),
            pltpu.VMEM_SHARED((N, D), _F32),
            pltpu.SemaphoreType.DMA,
            pltpu.SemaphoreType.DMA,
            pltpu.SemaphoreType.DMA,
            pltpu.SemaphoreType.DMA,
            pltpu.SemaphoreType.DMA,
            pltpu.SemaphoreType.DMA,
            pltpu.SemaphoreType.DMA,
            pltpu.SemaphoreType.DMA,
            pltpu.SemaphoreType.DMA,
        ],
    )
    def scatter(m_hbm, dst_hbm, out_hbm, di, mb, zbuf, acc,
                ms0, ms1, ms2, ms3, ss0, ss1, ss2, ss3, zs):
        cid = lax.axis_index("c")
        sid = lax.axis_index("s")
        wid = sid * NC + cid
        base = wid * epw
        ms = (ms0, ms1, ms2, ms3)
        ss = (ss0, ss1, ss2, ss3)

        def issue(ci, b4):
            pltpu.async_copy(m_hbm.at[pl.ds(base + ci * k, k)], mb.at[b4], ms[b4])
            pltpu.async_copy(dst_hbm.at[wid, ci], di.at[b4], ms[b4])

        for c0 in range(2):
            issue(c0, c0)

        def zrow(i, carry):
            for c in range(D // 16):
                zbuf[i, pl.ds(c * 16, 16)] = jnp.zeros((16,), _F32)
            return carry

        lax.fori_loop(0, _ZK, zrow, 0)

        nzj = (_NZ + NS - 1) // NS

        def zchunk(j, carry):
            cidx = j * NS + sid

            @pl.when(cidx < _NZ)
            def _():
                pltpu.async_copy(zbuf, acc.at[pl.ds(cidx * _ZK, _ZK)], zs)

            return carry

        lax.fori_loop(0, nzj, zchunk, 0)

        def zwait(j, carry):
            cidx = j * NS + sid

            @pl.when(cidx < _NZ)
            def _():
                pltpu.make_async_copy(zbuf, acc.at[pl.ds(0, _ZK)], zs).wait()

            return carry

        lax.fori_loop(0, nzj, zwait, 0)
        plsc.subcore_barrier()

        def swait(b4):
            pltpu.make_async_copy(mb.at[b4], acc.at[di.at[0]], ss[b4]).wait()

        def do_chunk(ci, b4, do_prefetch):
            pltpu.make_async_copy(m_hbm.at[pl.ds(0, k)], mb.at[b4], ms[b4]).wait()
            pltpu.make_async_copy(dst_hbm.at[wid, 0], di.at[b4], ms[b4]).wait()
            pltpu.async_copy(mb.at[b4], acc.at[di.at[b4]], ss[b4], add=True)
            if do_prefetch:
                nb = (b4 + 2) % 4

                @pl.when(ci >= 2)
                def _():
                    swait(nb)

                @pl.when(ci + 2 < nchunk)
                def _():
                    issue(ci + 2, nb)

        def body(j, carry):
            ci0 = j * 4
            for u in range(4):
                do_chunk(ci0 + u, u, do_prefetch=True)
            return carry

        lax.fori_loop(0, nchunk // 4, body, 0)

        for ci in range((nchunk // 4) * 4, nchunk):
            do_chunk(ci, ci % 4, do_prefetch=False)

        for b4 in (2, 3, 0):
            swait(b4)

        plsc.subcore_barrier()

        def wchunk(j, carry):
            cidx = j * NS + sid

            @pl.when(cidx < _NCH)
            def _():
                pltpu.async_copy(acc.at[pl.ds(cidx * _CH, _CH)],
                                 out_hbm.at[cid, pl.ds(cidx * _CH, _CH)], zs)

            return carry

        lax.fori_loop(0, (_NCH + NS - 1) // NS, wchunk, 0)

        def wwait(j, carry):
            cidx = j * NS + sid

            @pl.when(cidx < _NCH)
            def _():
                pltpu.make_async_copy(acc.at[pl.ds(0, _CH)],
                                      out_hbm.at[cid, pl.ds(0, _CH)], zs).wait()

            return carry

        lax.fori_loop(0, (_NCH + NS - 1) // NS, wwait, 0)

    return scatter


K = 80

_sc_gather = _make_gather(E, K)
_sc_scatter = _make_scatter(E, K)
_edge_mlp = _make_edge_mlp(E)


def kernel(x, x_attr, y, y_attr, edge_index, batches_seen,
           W_m1, b_m1, W_m2, b_m2, W_u1, b_u1, W_u2, b_u2, W_u3, b_u3):
    nch = (E // NW) // K
    src = edge_index[0].reshape(NW, nch, K)
    dst = edge_index[1].reshape(NW, nch, K)
    w1a = W_m1[:D]
    w1b = W_m1[D:]
    b1 = b_m1.reshape(1, D)
    b2 = b_m2.reshape(1, D)
    bu1 = b_u1.reshape(1, D)
    bu2 = b_u2.reshape(1, D)
    bu3 = b_u3.reshape(1, D)

    xx = x[0, -1]
    p, q = _node_in(xx, w1a, w1b, b1)

    outs = []
    for _ in range(4):
        h = _sc_gather(p, q, dst, src)
        m = _edge_mlp(h, W_m2, b2)
        agg = _sc_scatter(m, dst)
        xx, p, q = _update(xx, agg[0], agg[1],
                           W_u1, bu1, W_u2, bu2, W_u3, bu3, w1a, w1b, b1)
        outs.append(xx)

    out = jnp.stack(outs, axis=0)
    return out[None]


# speedup vs baseline: 1.0084x; 1.0084x over previous
import dataclasses
import functools

import jax
import jax.numpy as jnp
from jax import lax
from jax.experimental import pallas as pl
from jax.experimental.pallas import tpu as pltpu
from jax.experimental.pallas import tpu_sc as plsc

N = 10000
E = 320000
D = 128

NC = 2
NS = 16
NW = NC * NS

_F32 = jnp.float32
_PREC = lax.Precision.DEFAULT


def _node_in_body(x_ref, w1a_ref, w1b_ref, b1_ref, p_ref, q_ref):
    xb = x_ref[...]
    p_ref[...] = jnp.dot(xb, w1a_ref[...], preferred_element_type=_F32, precision=_PREC) + b1_ref[...]
    q_ref[...] = jnp.dot(xb, w1b_ref[...], preferred_element_type=_F32, precision=_PREC)


_node_in = pl.pallas_call(
    _node_in_body,
    grid=(5,),
    in_specs=[
        pl.BlockSpec((2000, D), lambda i: (i, 0)),
        pl.BlockSpec((D, D), lambda i: (0, 0)),
        pl.BlockSpec((D, D), lambda i: (0, 0)),
        pl.BlockSpec((1, D), lambda i: (0, 0)),
    ],
    out_specs=[
        pl.BlockSpec((2000, D), lambda i: (i, 0)),
        pl.BlockSpec((2000, D), lambda i: (i, 0)),
    ],
    out_shape=[
        jax.ShapeDtypeStruct((N, D), _F32),
        jax.ShapeDtypeStruct((N, D), _F32),
    ],
)


def _edge_mlp_body(h_ref, w2_ref, b2_ref, m_ref):
    h = h_ref[...]
    m_ref[...] = jnp.maximum(
        jnp.dot(h, w2_ref[...], preferred_element_type=_F32, precision=_PREC) + b2_ref[...], 0.0)


def _make_edge_mlp(e_rows):
    return pl.pallas_call(
        _edge_mlp_body,
        grid=(e_rows // 12800,),
        in_specs=[
            pl.BlockSpec((12800, D), lambda i: (i, 0)),
            pl.BlockSpec((D, D), lambda i: (0, 0)),
            pl.BlockSpec((1, D), lambda i: (0, 0)),
        ],
        out_specs=pl.BlockSpec((12800, D), lambda i: (i, 0)),
        out_shape=jax.ShapeDtypeStruct((e_rows, D), _F32),
    )


def _update_body(xx_ref, a0_ref, a1_ref,
                 wu1_ref, bu1_ref, wu2_ref, bu2_ref, wu3_ref, bu3_ref,
                 w1a_ref, w1b_ref, b1_ref, out_ref, p_ref, q_ref):
    agg = a0_ref[...] + a1_ref[...]
    u = jnp.maximum(
        jnp.dot(agg, wu1_ref[...], preferred_element_type=_F32, precision=_PREC) + bu1_ref[...], 0.0)
    u = jnp.maximum(
        jnp.dot(u, wu2_ref[...], preferred_element_type=_F32, precision=_PREC) + bu2_ref[...], 0.0)
    u = jnp.dot(u, wu3_ref[...], preferred_element_type=_F32, precision=_PREC) + bu3_ref[...]
    o = xx_ref[...] + u
    out_ref[...] = o
    p_ref[...] = jnp.dot(o, w1a_ref[...], preferred_element_type=_F32, precision=_PREC) + b1_ref[...]
    q_ref[...] = jnp.dot(o, w1b_ref[...], preferred_element_type=_F32, precision=_PREC)


_update = pl.pallas_call(
    _update_body,
    grid=(5,),
    in_specs=[pl.BlockSpec((2000, D), lambda i: (i, 0))] * 3
    + [pl.BlockSpec((D, D), lambda i: (0, 0)), pl.BlockSpec((1, D), lambda i: (0, 0))] * 3
    + [pl.BlockSpec((D, D), lambda i: (0, 0)),
       pl.BlockSpec((D, D), lambda i: (0, 0)),
       pl.BlockSpec((1, D), lambda i: (0, 0))],
    out_specs=[
        pl.BlockSpec((2000, D), lambda i: (i, 0)),
        pl.BlockSpec((2000, D), lambda i: (i, 0)),
        pl.BlockSpec((2000, D), lambda i: (i, 0)),
    ],
    out_shape=[
        jax.ShapeDtypeStruct((N, D), _F32),
        jax.ShapeDtypeStruct((N, D), _F32),
        jax.ShapeDtypeStruct((N, D), _F32),
    ],
)


_MESH = plsc.VectorSubcoreMesh(core_axis_name="c", subcore_axis_name="s")

_SC_CP = pltpu.CompilerParams()
if "needs_layout_passes" in pltpu.CompilerParams.__dataclass_fields__:
    _SC_CP = dataclasses.replace(_SC_CP, needs_layout_passes=False)


def _make_gather(e_edges, k):
    epw = e_edges // NW
    nchunk = epw // k
    assert nchunk == 125 and k % 8 == 0

    @functools.partial(
        pl.kernel,
        out_type=jax.ShapeDtypeStruct((e_edges, D), _F32),
        mesh=_MESH,
        compiler_params=_SC_CP,
        scratch_types=[
            pltpu.VMEM((nchunk, k), jnp.int32),
            pltpu.VMEM((nchunk, k), jnp.int32),
            pltpu.VMEM((3, k, D), _F32),
            pltpu.VMEM((3, k, D), _F32),
            pltpu.VMEM((2, k, D), _F32),
            pltpu.SemaphoreType.DMA,
            pltpu.SemaphoreType.DMA,
            pltpu.SemaphoreType.DMA,
            pltpu.SemaphoreType.DMA,
            pltpu.SemaphoreType.DMA,
        ],
    )
    def gather(p_hbm, q_hbm, dst_hbm, src_hbm, h_hbm,
               di, si, pb, qb, hb, gs0, gs1, gs2, ws0, ws1):
        wid = lax.axis_index("s") * NC + lax.axis_index("c")
        base = wid * epw
        gs = (gs0, gs1, gs2)
        ws = (ws0, ws1)

        pltpu.sync_copy(dst_hbm.at[wid], di)
        pltpu.sync_copy(src_hbm.at[wid], si)

        def issue(ci, b3):
            pltpu.async_copy(p_hbm.at[di.at[ci]], pb.at[b3], gs[b3])
            pltpu.async_copy(q_hbm.at[si.at[ci]], qb.at[b3], gs[b3])

        def gwait(b3):
            pltpu.make_async_copy(p_hbm.at[di.at[0]], pb.at[b3], gs[b3]).wait()
            pltpu.make_async_copy(q_hbm.at[si.at[0]], qb.at[b3], gs[b3]).wait()

        def wbwait(b2):
            pltpu.make_async_copy(hb.at[b2], h_hbm.at[pl.ds(0, k)], ws[b2]).wait()

        def do_chunk(ci, b3, b2, guard_wb, do_prefetch):
            gwait(b3)
            if guard_wb:
                @pl.when(ci >= 2)
                def _():
                    wbwait(b2)
            else:
                wbwait(b2)

            def row(i, c2):
                for c in range(D // 16):
                    sl = pl.ds(c * 16, 16)
                    hb[b2, i, sl] = jnp.maximum(pb[b3, i, sl] + qb[b3, i, sl], 0.0)
                return c2

            lax.fori_loop(0, k, row, 0)
            pltpu.async_copy(hb.at[b2], h_hbm.at[pl.ds(base + ci * k, k)], ws[b2])
            if do_prefetch:
                issue(ci + 3, b3)

        for c0 in range(3):
            issue(c0, c0)

        def body(j, carry):
            ci0 = j * 6
            for u in range(6):
                do_chunk(ci0 + u, u % 3, u % 2, guard_wb=True, do_prefetch=True)
            return carry

        lax.fori_loop(0, 120 // 6, body, 0)

        for ci in range(120, nchunk):
            do_chunk(ci, ci % 3, ci % 2, guard_wb=False,
                     do_prefetch=(ci + 3 < nchunk))

        wbwait(1)
        wbwait(0)

    return gather


_ZK = 40
_NZ = N // _ZK
_CH = 200
_NCH = N // _CH


def _make_scatter(e_edges, k):
    epw = e_edges // NW
    nchunk = epw // k
    assert nchunk == 125 and k % 8 == 0

    @functools.partial(
        pl.kernel,
        out_type=jax.ShapeDtypeStruct((NC, N, D), _F32),
        mesh=_MESH,
        compiler_params=_SC_CP,
        scratch_types=[
            pltpu.VMEM((4, k), jnp.int32),
            pltpu.VMEM((4, k, D), _F32),
            pltpu.VMEM((_ZK, D), _F32),
            pltpu.VMEM_SHARED((N, D), _F32),
            pltpu.SemaphoreType.DMA,
            pltpu.SemaphoreType.DMA,
            pltpu.SemaphoreType.DMA,
            pltpu.SemaphoreType.DMA,
            pltpu.SemaphoreType.DMA,
            pltpu.SemaphoreType.DMA,
            pltpu.SemaphoreType.DMA,
            pltpu.SemaphoreType.DMA,
            pltpu.SemaphoreType.DMA,
        ],
    )
    def scatter(m_hbm, dst_hbm, out_hbm, di, mb, zbuf, acc,
                ms0, ms1, ms2, ms3, ss0, ss1, ss2, ss3, zs):
        cid = lax.axis_index("c")
        sid = lax.axis_index("s")
        wid = sid * NC + cid
        base = wid * epw
        ms = (ms0, ms1, ms2, ms3)
        ss = (ss0, ss1, ss2, ss3)

        def issue(ci, b4):
            pltpu.async_copy(m_hbm.at[pl.ds(base + ci * k, k)], mb.at[b4], ms[b4])
            pltpu.async_copy(dst_hbm.at[wid, ci], di.at[b4], ms[b4])

        for c0 in range(2):
            issue(c0, c0)

        def zrow(i, carry):
            for c in range(D // 16):
                zbuf[i, pl.ds(c * 16, 16)] = jnp.zeros((16,), _F32)
            return carry

        lax.fori_loop(0, _ZK, zrow, 0)

        nzj = (_NZ + NS - 1) // NS

        def zchunk(j, carry):
            cidx = j * NS + sid

            @pl.when(cidx < _NZ)
            def _():
                pltpu.async_copy(zbuf, acc.at[pl.ds(cidx * _ZK, _ZK)], zs)

            return carry

        lax.fori_loop(0, nzj, zchunk, 0)

        def zwait(j, carry):
            cidx = j * NS + sid

            @pl.when(cidx < _NZ)
            def _():
                pltpu.make_async_copy(zbuf, acc.at[pl.ds(0, _ZK)], zs).wait()

            return carry

        lax.fori_loop(0, nzj, zwait, 0)
        plsc.subcore_barrier()

        def swait(b4):
            pltpu.make_async_copy(mb.at[b4], acc.at[di.at[0]], ss[b4]).wait()

        def do_chunk(ci, b4, do_prefetch):
            pltpu.make_async_copy(m_hbm.at[pl.ds(0, k)], mb.at[b4], ms[b4]).wait()
            pltpu.make_async_copy(dst_hbm.at[wid, 0], di.at[b4], ms[b4]).wait()
            pltpu.async_copy(mb.at[b4], acc.at[di.at[b4]], ss[b4], add=True)
            if do_prefetch:
                nb = (b4 + 2) % 4

                @pl.when(ci >= 2)
                def _():
                    swait(nb)

                @pl.when(ci + 2 < nchunk)
                def _():
                    issue(ci + 2, nb)

        def body(j, carry):
            ci0 = j * 4
            for u in range(4):
                do_chunk(ci0 + u, u, do_prefetch=True)
            return carry

        lax.fori_loop(0, nchunk // 4, body, 0)

        for ci in range((nchunk // 4) * 4, nchunk):
            do_chunk(ci, ci % 4, do_prefetch=False)

        for b4 in (2, 3, 0):
            swait(b4)

        plsc.subcore_barrier()

        def wchunk(j, carry):
            cidx = j * NS + sid

            @pl.when(cidx < _NCH)
            def _():
                pltpu.async_copy(acc.at[pl.ds(cidx * _CH, _CH)],
                                 out_hbm.at[cid, pl.ds(cidx * _CH, _CH)], zs)

            return carry

        lax.fori_loop(0, (_NCH + NS - 1) // NS, wchunk, 0)

        def wwait(j, carry):
            cidx = j * NS + sid

            @pl.when(cidx < _NCH)
            def _():
                pltpu.make_async_copy(acc.at[pl.ds(0, _CH)],
                                      out_hbm.at[cid, pl.ds(0, _CH)], zs).wait()

            return carry

        lax.fori_loop(0, (_NCH + NS - 1) // NS, wwait, 0)

    return scatter


K = 80

_sc_gather = _make_gather(E, K)
_sc_scatter = _make_scatter(E, K)
_edge_mlp = _make_edge_mlp(E)


def kernel(x, x_attr, y, y_attr, edge_index, batches_seen,
           W_m1, b_m1, W_m2, b_m2, W_u1, b_u1, W_u2, b_u2, W_u3, b_u3):
    nch = (E // NW) // K
    src = edge_index[0].reshape(NW, nch, K)
    dst = edge_index[1].reshape(NW, nch, K)
    w1a = W_m1[:D]
    w1b = W_m1[D:]
    b1 = b_m1.reshape(1, D)
    b2 = b_m2.reshape(1, D)
    bu1 = b_u1.reshape(1, D)
    bu2 = b_u2.reshape(1, D)
    bu3 = b_u3.reshape(1, D)

    xx = x[0, -1]
    p, q = _node_in(xx, w1a, w1b, b1)

    outs = []
    for _ in range(4):
        h = _sc_gather(p, q, dst, src)
        m = _edge_mlp(h, W_m2, b2)
        agg = _sc_scatter(m, dst)
        xx, p, q = _update(xx, agg[0], agg[1],
                           W_u1, bu1, W_u2, bu2, W_u3, bu3, w1a, w1b, b1)
        outs.append(xx)

    out = jnp.stack(outs, axis=0)
    return out[None]
